# baseline (speedup 1.0000x reference)
import functools

import jax
import jax.numpy as jnp
from jax import lax
from jax.experimental import pallas as pl
from jax.experimental.pallas import tpu as pltpu
from jax.experimental.pallas import tpu_sc as plsc

_DIM = 32
_LANES = 16
_SUPER = 128


def _make_kernel(batch):
    info = plsc.get_sparse_core_info()
    nc, ns = info.num_cores, info.num_subcores
    nw = nc * ns
    b_per_w = batch // nw
    n_chunks = b_per_w // 128

    mesh = plsc.VectorSubcoreMesh(core_axis_name="c", subcore_axis_name="s")

    @functools.partial(
        pl.kernel,
        mesh=mesh,
        out_type=jax.ShapeDtypeStruct((nw, _LANES), jnp.float32),
        compiler_params=pltpu.CompilerParams(needs_layout_passes=False),
        scratch_types=[
            pltpu.VMEM((n_chunks, 128), jnp.int32),
            pltpu.VMEM((n_chunks, 128), jnp.int32),
            pltpu.VMEM((b_per_w,), jnp.int32),
            pltpu.VMEM((b_per_w,), jnp.int32),
            pltpu.VMEM((b_per_w,), jnp.float32),
            pltpu.VMEM((128, _SUPER), jnp.float32),
            pltpu.VMEM((128, _SUPER), jnp.float32),
            pltpu.VMEM((128, _SUPER), jnp.float32),
            pltpu.VMEM((128, _SUPER), jnp.float32),
            pltpu.VMEM((_LANES,), jnp.float32),
            pltpu.SemaphoreType.DMA,
        ],
    )
    def sc_kernel(lsup_hbm, rsup_hbm, lsub_hbm, rsub_hbm, cov_hbm,
                  table_hbm, out_hbm,
                  lsup_v, rsup_v, lsub_v, rsub_v, cov_v,
                  lrows_a, lrows_b, rrows_a, rrows_b, loss_v, sem):
        wid = lax.axis_index("s") * nc + lax.axis_index("c")

        pltpu.sync_copy(lsup_hbm.at[wid], lsup_v)
        pltpu.sync_copy(rsup_hbm.at[wid], rsup_v)
        pltpu.sync_copy(lsub_hbm.at[wid], lsub_v)
        pltpu.sync_copy(rsub_hbm.at[wid], rsub_v)
        pltpu.sync_copy(cov_hbm.at[wid], cov_v)

        lbufs = (lrows_a, lrows_b)
        rbufs = (rrows_a, rrows_b)

        def fire(j):
            return (pltpu.async_copy(table_hbm.at[lsup_v.at[j]],
                                     lbufs[j % 2], sem),
                    pltpu.async_copy(table_hbm.at[rsup_v.at[j]],
                                     rbufs[j % 2], sem))

        lane = lax.iota(jnp.int32, _LANES)
        groups_per_chunk = 128 // _LANES

        def make_group_body(lbuf, rbuf, j):
            def group_body(g, loss):
                sl = pl.ds(j * 128 + g * _LANES, _LANES)
                row = g * _LANES + lane
                loff = lsub_v[sl]
                roff = rsub_v[sl]
                acc = jnp.zeros((_LANES,), jnp.float32)
                for c in range(_DIM):
                    lv = plsc.load_gather(lbuf, [row, loff + c])
                    rv = plsc.load_gather(rbuf, [row, roff + c])
                    acc = acc + lv * rv
                d = acc - cov_v[sl]
                return loss + d * d
            return group_body

        inflight = [fire(0), fire(1)]
        loss = jnp.zeros((_LANES,), jnp.float32)
        for j in range(n_chunks):
            for cp in inflight.pop(0):
                cp.wait()
            loss = lax.fori_loop(
                0, groups_per_chunk,
                make_group_body(lbufs[j % 2], rbufs[j % 2], j), loss)
            if j + 2 < n_chunks:
                inflight.append(fire(j + 2))
        loss_v[...] = loss
        pltpu.sync_copy(loss_v, out_hbm.at[wid])

    return nw, n_chunks, sc_kernel


def kernel(left, right, covariances, table):
    batch = left.shape[0]
    size, dim = table.shape
    nw, n_chunks, sc_kernel = _make_kernel(batch)
    emb_per_super = _SUPER // dim
    table128 = table.reshape(size // emb_per_super, _SUPER)
    left = left.astype(jnp.int32)
    right = right.astype(jnp.int32)
    lsup = (left // emb_per_super).reshape(nw, n_chunks, 128)
    rsup = (right // emb_per_super).reshape(nw, n_chunks, 128)
    lsub = ((left % emb_per_super) * dim).reshape(nw, batch // nw)
    rsub = ((right % emb_per_super) * dim).reshape(nw, batch // nw)
    cov2 = covariances.reshape(nw, batch // nw)
    partials = sc_kernel(lsup, rsup, lsub, rsub, cov2, table128)
    return jnp.sum(partials) / batch
